# Initial kernel scaffold; baseline (speedup 1.0000x reference)
import jax
import jax.numpy as jnp
from jax.experimental import pallas as pl


def kernel(x, edge_index, W1, b1, W2, b2):
    raise NotImplementedError("write your pallas kernel here")


# speedup vs baseline: 1.9297x; 1.9297x over previous
import functools

import jax
import jax.numpy as jnp
from jax.experimental import pallas as pl
from jax.experimental.pallas import tpu as pltpu

N = 10000
E = 160000
NP = 10240
BLK = 256
NBLK = NP // BLK


def _p3_body(xs_ref, a0_ref, a1_ref, db_ref, cb_ref,
             w1_ref, b1_ref, w2_ref, b2_ref, out_ref, msum):
    i = pl.program_id(0)

    @pl.when(i == 0)
    def _():
        msum[...] = jnp.zeros_like(msum)

    db = db_ref[...]
    agg = db * (a0_ref[...] + a1_ref[...] + xs_ref[...])
    h = jax.lax.dot_general(agg, w1_ref[...], (((1,), (1,)), ((), ())),
                            precision=jax.lax.Precision.HIGHEST,
                            preferred_element_type=jnp.float32)
    h = jnp.maximum(h + b1_ref[...], 0.0)
    msum[...] += jnp.sum(cb_ref[...] * h, axis=0, keepdims=True)

    @pl.when(i == NBLK - 1)
    def _():
        m = msum[...] * (1.0 / N)
        out_ref[...] = jax.lax.dot_general(
            m, w2_ref[...], (((1,), (1,)), ((), ())),
            precision=jax.lax.Precision.HIGHEST,
            preferred_element_type=jnp.float32) + b2_ref[...]


@jax.jit
def _p3(xs, a0, a1, db, cb, W1, b1, W2, b2):
    row_spec = pl.BlockSpec((BLK, 128), lambda i: (i, 0))
    full = lambda shape: pl.BlockSpec(shape, lambda i: (0, 0))
    return pl.pallas_call(
        _p3_body,
        grid=(NBLK,),
        in_specs=[row_spec, row_spec, row_spec, row_spec, row_spec,
                  full((128, 128)), full((1, 128)), full((64, 128)), full((1, 64))],
        out_specs=full((1, 64)),
        out_shape=jax.ShapeDtypeStruct((1, 64), jnp.float32),
        scratch_shapes=[pltpu.VMEM((1, 128), jnp.float32)],
    )(xs, a0, a1, db, cb, W1, b1, W2, b2)


def kernel(x, edge_index, W1, b1, W2, b2):
    src = edge_index[0]
    dst = edge_index[1]

    deg = jax.ops.segment_sum(jnp.ones((E,), jnp.float32), dst, num_segments=N) + 1.0
    dis = deg ** -0.5
    xs = dis[:, None] * x
    acc = jax.ops.segment_sum(xs[src], dst, num_segments=N)
    s = jax.ops.segment_sum(dis[dst], src, num_segments=N)

    c = dis * (s + dis)
    pad = NP - N
    xs_p = jnp.pad(xs, ((0, pad), (0, 0)))
    a0_p = jnp.pad(acc, ((0, pad), (0, 0)))
    a1_p = jnp.zeros((NP, 128), jnp.float32)
    db_p = jnp.broadcast_to(jnp.pad(dis, (0, pad))[:, None], (NP, 128))
    cb_p = jnp.broadcast_to(jnp.pad(c, (0, pad))[:, None], (NP, 128))
    return _p3(xs_p, a0_p, a1_p, db_p, cb_p,
               W1, b1.reshape(1, 128), W2, b2.reshape(1, 64))


# speedup vs baseline: 14.8295x; 7.6850x over previous
import functools

import jax
import jax.numpy as jnp
from jax import lax
from jax.experimental import pallas as pl
from jax.experimental.pallas import tpu as pltpu
from jax.experimental.pallas import tpu_sc as plsc

N = 10000
E = 160000
NP = 10240
NR = NP // 128
BLK = 256
NBLK = NP // BLK
NTILES = 32
EPT = E // NTILES
CH = 128
NCH = EPT // CH
TAIL = EPT - NCH * CH

_mesh = plsc.VectorSubcoreMesh(core_axis_name="c", subcore_axis_name="s",
                               num_cores=2, num_subcores=16)
_sc_params = pltpu.CompilerParams(needs_layout_passes=False)


def _zero_1d(ref, n):
    def body(i, _):
        ref[pl.ds(i * 16, 16)] = jnp.zeros((16,), jnp.float32)
        return 0
    lax.fori_loop(0, n // 16, body, 0)


def _reduce16(hist, sh, sum_buf, res_buf, out_hbm, c, s):
    seg = NP // 16
    pltpu.sync_copy(hist, sh.at[s])
    plsc.subcore_barrier()
    pltpu.sync_copy(sh.at[:, pl.ds(s * seg, seg)], sum_buf)

    def col(i, _):
        acc = jnp.zeros((16,), jnp.float32)
        def row(r, a):
            return a + sum_buf[r, pl.ds(i * 16, 16)]
        acc = lax.fori_loop(0, 16, row, acc)
        res_buf[pl.ds(i * 16, 16)] = acc
        return 0

    lax.fori_loop(0, seg // 16, col, 0)
    pltpu.sync_copy(res_buf, out_hbm.at[c, pl.ds(s * seg, seg)])


@functools.partial(
    pl.kernel,
    out_type=jax.ShapeDtypeStruct((2, NP), jnp.float32),
    mesh=_mesh,
    compiler_params=_sc_params,
    scratch_types=[
        pltpu.VMEM((CH,), jnp.int32),
        pltpu.VMEM((NP,), jnp.float32),
        pltpu.VMEM((16, NP // 16), jnp.float32),
        pltpu.VMEM((NP // 16,), jnp.float32),
        pltpu.VMEM_SHARED((16, NP), jnp.float32),
    ],
)
def _p1(dst_hbm, out_hbm, idx_buf, hist, sum_buf, res_buf, sh):
    c = lax.axis_index("c")
    s = lax.axis_index("s")
    wid = s * 2 + c
    base = wid * EPT

    _zero_1d(hist, NP)
    ones = jnp.ones((16,), jnp.float32)

    def chunk(j, _):
        pltpu.sync_copy(dst_hbm.at[pl.ds(base + j * CH, CH)], idx_buf)
        def sub(k, _2):
            plsc.addupdate_scatter(hist, [idx_buf[pl.ds(k * 16, 16)]], ones)
            return 0
        lax.fori_loop(0, CH // 16, sub, 0)
        return 0

    lax.fori_loop(0, NCH, chunk, 0)
    pltpu.sync_copy(dst_hbm.at[pl.ds(base + NCH * CH, TAIL)], idx_buf.at[pl.ds(0, TAIL)])
    plsc.addupdate_scatter(hist, [idx_buf[pl.ds(0, 16)]], ones,
                           mask=lax.iota(jnp.int32, 16) < TAIL)

    _reduce16(hist, sh, sum_buf, res_buf, out_hbm, c, s)


@functools.partial(
    pl.kernel,
    out_type=jax.ShapeDtypeStruct((2 * NP, 128), jnp.float32),
    mesh=_mesh,
    compiler_params=_sc_params,
    scratch_types=[
        pltpu.VMEM((CH,), jnp.int32),
        pltpu.VMEM((CH,), jnp.int32),
        pltpu.VMEM((CH, 128), jnp.float32),
        pltpu.VMEM((TAIL,), jnp.int32),
        pltpu.VMEM((TAIL,), jnp.int32),
        pltpu.VMEM((TAIL, 128), jnp.float32),
        pltpu.VMEM_SHARED((NP, 128), jnp.float32),
        pltpu.SemaphoreType.DMA,
    ],
)
def _p2(xs_hbm, src_hbm, dst_hbm, acc_out,
        src_buf, dst_buf, rows, src_tail, dst_tail, rows_tail,
        acc_sh, gsem):
    c = lax.axis_index("c")
    s = lax.axis_index("s")
    wid = s * 2 + c
    base = wid * EPT
    tb = s * (NP // 16)

    def zrow(i, _):
        rows[lax.shift_right_logical(i, 3),
             pl.ds(lax.mul(lax.bitwise_and(i, 7), 16), 16)] = (
                 jnp.zeros((16,), jnp.float32))
        return 0
    lax.fori_loop(0, CH * 8, zrow, 0)
    for k in range(NP // 16 // CH):
        pltpu.sync_copy(rows, acc_sh.at[pl.ds(tb + k * CH, CH)])
    plsc.subcore_barrier()

    def chunk(j, _):
        off = base + j * CH
        pltpu.sync_copy(src_hbm.at[pl.ds(off, CH)], src_buf)
        pltpu.sync_copy(dst_hbm.at[pl.ds(off, CH)], dst_buf)
        pltpu.async_copy(xs_hbm.at[src_buf], rows, gsem).wait()
        pltpu.sync_copy(rows, acc_sh.at[dst_buf], add=True)
        return 0

    lax.fori_loop(0, NCH, chunk, 0)

    off = base + NCH * CH
    pltpu.sync_copy(src_hbm.at[pl.ds(off, TAIL)], src_tail)
    pltpu.sync_copy(dst_hbm.at[pl.ds(off, TAIL)], dst_tail)
    pltpu.async_copy(xs_hbm.at[src_tail], rows_tail, gsem).wait()
    pltpu.sync_copy(rows_tail, acc_sh.at[dst_tail], add=True)

    plsc.subcore_barrier()
    pltpu.sync_copy(acc_sh.at[pl.ds(tb, NP // 16)],
                    acc_out.at[pl.ds(c * NP + tb, NP // 16)])


@functools.partial(
    pl.kernel,
    out_type=jax.ShapeDtypeStruct((2, NP), jnp.float32),
    mesh=_mesh,
    compiler_params=_sc_params,
    scratch_types=[
        pltpu.VMEM((CH,), jnp.int32),
        pltpu.VMEM((CH,), jnp.int32),
        pltpu.VMEM((NP,), jnp.float32),
        pltpu.VMEM((NP,), jnp.float32),
        pltpu.VMEM((16, NP // 16), jnp.float32),
        pltpu.VMEM((NP // 16,), jnp.float32),
        pltpu.VMEM_SHARED((16, NP), jnp.float32),
    ],
)
def _p2b(src_hbm, dst_hbm, dis_hbm, out_hbm,
         src_buf, dst_buf, dis_t, hist, sum_buf, res_buf, sh):
    c = lax.axis_index("c")
    s = lax.axis_index("s")
    wid = s * 2 + c
    base = wid * EPT

    _zero_1d(hist, NP)
    pltpu.sync_copy(dis_hbm, dis_t)

    def s_update(k, mask=None):
        dv = dst_buf[pl.ds(k * 16, 16)]
        vals = plsc.load_gather(dis_t, [dv])
        sv = src_buf[pl.ds(k * 16, 16)]
        plsc.addupdate_scatter(hist, [sv], vals, mask=mask)

    def chunk(j, _):
        off = base + j * CH
        pltpu.sync_copy(src_hbm.at[pl.ds(off, CH)], src_buf)
        pltpu.sync_copy(dst_hbm.at[pl.ds(off, CH)], dst_buf)
        def sub(k, _2):
            s_update(k)
            return 0
        lax.fori_loop(0, CH // 16, sub, 0)
        return 0

    lax.fori_loop(0, NCH, chunk, 0)
    off = base + NCH * CH
    pltpu.sync_copy(src_hbm.at[pl.ds(off, TAIL)], src_buf.at[pl.ds(0, TAIL)])
    pltpu.sync_copy(dst_hbm.at[pl.ds(off, TAIL)], dst_buf.at[pl.ds(0, TAIL)])
    s_update(0, mask=lax.iota(jnp.int32, 16) < TAIL)

    _reduce16(hist, sh, sum_buf, res_buf, out_hbm, c, s)


def _p15_body(x_ref, db_ref, xs_ref):
    xs_ref[...] = x_ref[...] * db_ref[...]


def _p15(x_p, db):
    spec = pl.BlockSpec((BLK, 128), lambda i: (i, 0))
    return pl.pallas_call(
        _p15_body,
        grid=(NBLK,),
        in_specs=[spec, spec],
        out_specs=spec,
        out_shape=jax.ShapeDtypeStruct((NP, 128), jnp.float32),
    )(x_p, db)


def _p3_body(xs_ref, a0_ref, a1_ref, db_ref, cb_ref,
             w1_ref, b1_ref, w2_ref, b2_ref, out_ref, msum):
    i = pl.program_id(0)

    @pl.when(i == 0)
    def _():
        msum[...] = jnp.zeros_like(msum)

    db = db_ref[...]
    agg = db * (a0_ref[...] + a1_ref[...] + xs_ref[...])
    h = lax.dot_general(agg, w1_ref[...], (((1,), (1,)), ((), ())),
                        precision=lax.Precision.HIGHEST,
                        preferred_element_type=jnp.float32)
    h = jnp.maximum(h + b1_ref[...], 0.0)
    msum[...] += jnp.sum(cb_ref[...] * h, axis=0, keepdims=True)

    @pl.when(i == NBLK - 1)
    def _():
        m = msum[...] * (1.0 / N)
        out_ref[...] = lax.dot_general(
            m, w2_ref[...], (((1,), (1,)), ((), ())),
            precision=lax.Precision.HIGHEST,
            preferred_element_type=jnp.float32) + b2_ref[...]


def _p3(xs, a0, a1, db, cb, W1, b1, W2, b2):
    row_spec = pl.BlockSpec((BLK, 128), lambda i: (i, 0))
    full = lambda shape: pl.BlockSpec(shape, lambda i: (0, 0))
    return pl.pallas_call(
        _p3_body,
        grid=(NBLK,),
        in_specs=[row_spec, row_spec, row_spec, row_spec, row_spec,
                  full((128, 128)), full((1, 128)), full((64, 128)), full((1, 64))],
        out_specs=full((1, 64)),
        out_shape=jax.ShapeDtypeStruct((1, 64), jnp.float32),
        scratch_shapes=[pltpu.VMEM((1, 128), jnp.float32)],
    )(xs, a0, a1, db, cb, W1, b1, W2, b2)


def kernel(x, edge_index, W1, b1, W2, b2):
    src = edge_index[0]
    dst = edge_index[1]

    degp = _p1(dst)
    deg = degp[0] + degp[1] + 1.0
    dis_flat = deg ** -0.5
    db = jnp.broadcast_to(dis_flat[:, None], (NP, 128))

    x_p = jnp.pad(x, ((0, NP - N), (0, 0)))
    xs = _p15(x_p, db)

    accf = _p2(xs, src, dst)
    sf = _p2b(src, dst, dis_flat)
    s_flat = sf[0] + sf[1]
    c = dis_flat * (s_flat + dis_flat)
    c = jnp.where(lax.iota(jnp.int32, NP) < N, c, 0.0)
    cb = jnp.broadcast_to(c[:, None], (NP, 128))

    return _p3(xs, accf[:NP], accf[NP:], db, cb,
               W1, b1.reshape(1, 128), W2, b2.reshape(1, 64))
